# Initial kernel scaffold; baseline (speedup 1.0000x reference)
import jax
import jax.numpy as jnp
from jax.experimental import pallas as pl


def kernel(center, context, neg_context, in_embed, out_embed):
    raise NotImplementedError("write your pallas kernel here")


# speedup vs baseline: 1.6172x; 1.6172x over previous
import functools

import jax
import jax.numpy as jnp
from jax import lax
from jax.experimental import pallas as pl
from jax.experimental.pallas import tpu as pltpu
from jax.experimental.pallas import tpu_sc as plsc

V = 1000000
D = 64
B = 16384
C = 4
K = 5
NT = 1 + K

_info = plsc.get_sparse_core_info()
NC = _info.num_cores
NS = _info.num_subcores
L = _info.num_lanes
NW = NC * NS
B_PER_W = B // NW
NCHUNK = 64
N_CHUNKS = B_PER_W // NCHUNK


def _sc_scores_kernel(ctx_idx_hbm, out_idx_hbm, in_embed_hbm, out_embed_hbm,
                      scores_hbm,
                      ctx_idx_v, out_idx_v, ctx_rows_v, out_rows_v, cm_t_v,
                      scores_v, sem_c, sem_o):
    wid = lax.axis_index("s") * NC + lax.axis_index("c")
    base = wid * B_PER_W
    iota = jnp.arange(L, dtype=jnp.int32)
    for chunk in range(N_CHUNKS):
        rb = base + chunk * NCHUNK
        pltpu.sync_copy(ctx_idx_hbm.at[pl.ds(rb * C, NCHUNK * C)], ctx_idx_v)
        pltpu.sync_copy(out_idx_hbm.at[pl.ds(rb * NT, NCHUNK * NT)], out_idx_v)
        cps = []
        for g in range(NCHUNK * C // 128):
            cps.append(pltpu.async_copy(
                in_embed_hbm.at[ctx_idx_v.at[pl.ds(g * 128, 128)]],
                ctx_rows_v.at[pl.ds(g * 128, 128)], sem_c))
        for g in range(NCHUNK * NT // 128):
            cps.append(pltpu.async_copy(
                out_embed_hbm.at[out_idx_v.at[pl.ds(g * 128, 128)]],
                out_rows_v.at[pl.ds(g * 128, 128)], sem_o))
        for cp in cps:
            cp.wait()

        def mean_body(b, carry):
            r0 = C * b
            colb = jnp.full((L,), b, dtype=jnp.int32)
            for m in range(D // L):
                s = pl.ds(m * L, L)
                v = (ctx_rows_v[r0, s] + ctx_rows_v[r0 + 1, s]
                     + ctx_rows_v[r0 + 2, s] + ctx_rows_v[r0 + 3, s])
                plsc.store_scatter(cm_t_v, [iota + (m * L), colb],
                                   v * (1.0 / C))
            return carry

        lax.fori_loop(0, NCHUNK, mean_body, 0)

        for g in range(NCHUNK // L):
            b0 = g * L
            rows = [(iota + b0) * NT + t for t in range(NT)]

            def dot_body(d, accs):
                cm = cm_t_v[d, pl.ds(b0, L)]
                cold = jnp.full((L,), d, dtype=jnp.int32)
                return tuple(
                    accs[t] + cm * plsc.load_gather(out_rows_v, [rows[t], cold])
                    for t in range(NT))

            accs = lax.fori_loop(
                0, D, dot_body,
                tuple(jnp.zeros((L,), jnp.float32) for _ in range(NT)))
            for t in range(NT):
                scores_v[t, pl.ds(b0, L)] = accs[t]

        for t in range(NT):
            pltpu.sync_copy(scores_v.at[t], scores_hbm.at[t, pl.ds(rb, NCHUNK)])


@jax.jit
def _sc_scores(ctx_idx, out_idx, in_embed, out_embed):
    mesh = plsc.VectorSubcoreMesh(core_axis_name="c", subcore_axis_name="s")
    f = functools.partial(
        pl.kernel, mesh=mesh,
        out_type=jax.ShapeDtypeStruct((NT, B), jnp.float32),
        scratch_types=[
            pltpu.VMEM((NCHUNK * C,), jnp.int32),
            pltpu.VMEM((NCHUNK * NT,), jnp.int32),
            pltpu.VMEM((NCHUNK * C, D), jnp.float32),
            pltpu.VMEM((NCHUNK * NT, D), jnp.float32),
            pltpu.VMEM((D, NCHUNK), jnp.float32),
            pltpu.VMEM((NT, NCHUNK), jnp.float32),
            pltpu.SemaphoreType.DMA,
            pltpu.SemaphoreType.DMA,
        ],
        compiler_params=pltpu.CompilerParams(
            needs_layout_passes=False, use_tc_tiling_on_sc=False),
    )(_sc_scores_kernel)
    return f(ctx_idx, out_idx, in_embed, out_embed)


def _loss_body(s_ref, o_ref):
    x = s_ref[...]
    is_pos = lax.broadcasted_iota(jnp.int32, x.shape, 0) == 0
    y = jnp.where(is_pos, -x, x)
    sp = jnp.maximum(y, 0.0) + jnp.log(1.0 + jnp.exp(-jnp.abs(y)))
    o_ref[0, 0] = jnp.sum(sp) * (1.0 / B)


@jax.jit
def _tc_loss(scores):
    return pl.pallas_call(
        _loss_body,
        out_shape=jax.ShapeDtypeStruct((1, 1), jnp.float32),
        out_specs=pl.BlockSpec(memory_space=pltpu.SMEM),
    )(scores)


def kernel(center, context, neg_context, in_embed, out_embed):
    ctx_idx = context.astype(jnp.int32).reshape(B * C)
    out_idx = jnp.concatenate(
        [center.astype(jnp.int32), neg_context.astype(jnp.int32)],
        axis=1).reshape(B * NT)
    scores = _sc_scores(ctx_idx, out_idx, in_embed, out_embed)
    loss = _tc_loss(scores)
    return loss[0, 0]
